# Initial kernel scaffold; baseline (speedup 1.0000x reference)
import jax
import jax.numpy as jnp
from jax.experimental import pallas as pl


def kernel(encode_andr_channel, encode_app_id, encode_device_model, encode_os_version, encode_dvce_manufacturer, encode_event_sub_type, collector_hour, collector_minute, emb_encode_andr_channel, emb_encode_app_id, emb_encode_device_model, emb_encode_os_version, emb_encode_dvce_manufacturer, emb_encode_event_sub_type, emb_collector_hour, emb_collector_minute, edge_index, W1, b1, W2, b2, W3, b3, W4, b4, W5, b5):
    raise NotImplementedError("write your pallas kernel here")


# speedup vs baseline: 9.1703x; 9.1703x over previous
import functools

import jax
import jax.numpy as jnp
from jax import lax
from jax.experimental import pallas as pl
from jax.experimental.pallas import tpu as pltpu
from jax.experimental.pallas import tpu_sc as plsc

_F32 = jnp.float32
_N = 100000
_E = 1600000
_NTILES = 32
_EPT = _E // _NTILES
_EBLK = 400
_NEB = _EPT // _EBLK
_NPS = _N // 16
_ZR = 250
_VS = [100, 5000, 2000, 50, 200, 50, 24, 60]
_DS = [8, 8, 16, 10, 10, 8, 8, 8]
_OFF = [0, 8, 16, 32, 42, 52, 60, 68]
_HID = 48
_OUT = 16
_NBLK = 200
_NBLK_N = _N // _NBLK
_NB_FULL = 16

_mesh = plsc.VectorSubcoreMesh(
    core_axis_name="c", subcore_axis_name="s", num_cores=2, num_subcores=16)
_SC_PARAMS = pltpu.CompilerParams(use_tc_tiling_on_sc=False)


def _wid_cid_sid():
  cid = lax.axis_index("c")
  sid = lax.axis_index("s")
  return cid * 16 + sid, cid, sid


def _zero_fill(ref, rows):
  def body(r, _):
    ref[r, :] = jnp.zeros((16,), _F32)
    return 0
  lax.fori_loop(0, rows, body, 0)


def _zero_acc(accsh, zbuf, sid):
  def body(r, _):
    pltpu.sync_copy(zbuf, accsh.at[pl.ds(sid * _NPS + r * _ZR, _ZR)])
    return 0
  lax.fori_loop(0, 25, body, 0)


def _copy_out(accsh, dst_at, sid):
  @pl.when(sid < 15)
  def _():
    off = pl.multiple_of(sid * 6256, 8)
    pltpu.sync_copy(accsh.at[pl.ds(off, 6256)], dst_at(off, 6256))

  @pl.when(sid == 15)
  def _():
    pltpu.sync_copy(accsh.at[pl.ds(93840, 6160)], dst_at(93840, 6160))


def _embed_deg_body(*refs):
  idxs = refs[0:8]
  tabs = refs[8:16]
  dstr = refs[16]
  h1o = refs[17]
  dego = refs[18]
  ib, gbuf, acc, ones, eb, zbuf, accsh, sem = refs[19:]
  wid, cid, sid = _wid_cid_sid()

  _zero_fill(zbuf, _ZR)
  def ones_body(r, _):
    ones[r, :] = jnp.full((16,), 1.0, _F32)
    return 0
  lax.fori_loop(0, _EBLK, ones_body, 0)

  _zero_acc(accsh, zbuf, sid)
  plsc.subcore_barrier()

  def deg_body(j, _):
    base = wid * _EPT + j * _EBLK
    pltpu.sync_copy(dstr.at[pl.ds(base, _EBLK)], eb)
    pltpu.sync_copy(ones, accsh.at[eb], add=True)
    return 0
  lax.fori_loop(0, _NEB, deg_body, 0)
  plsc.subcore_barrier()
  _copy_out(accsh, lambda off, sz: dego.at[cid, pl.ds(off, sz)], sid)

  nblk = jnp.where(wid < (_NBLK_N - (_NB_FULL - 1) * _NTILES),
                   _NB_FULL, _NB_FULL - 1)

  def emb_body(b, _):
    base = (wid + _NTILES * b) * _NBLK
    pltpu.sync_copy(idxs[0].at[pl.ds(base, _NBLK)], ib)
    pltpu.async_copy(tabs[0].at[ib], acc, sem).wait()
    for f in range(1, 8):
      pltpu.sync_copy(idxs[f].at[pl.ds(base, _NBLK)], ib)
      pltpu.async_copy(tabs[f].at[ib], gbuf, sem).wait()
      def add_body(r, _):
        for c in range(3):
          plsc.addupdate(acc.at[r, pl.ds(c * 16, 16)],
                         gbuf[r, pl.ds(c * 16, 16)])
        return 0
      lax.fori_loop(0, _NBLK, add_body, 0)
    pltpu.sync_copy(acc, h1o.at[pl.ds(base, _NBLK)])
    return 0
  lax.fori_loop(0, nblk, emb_body, 0)


_embed_deg = functools.partial(
    pl.kernel,
    out_type=[jax.ShapeDtypeStruct((_N, _HID), _F32),
              jax.ShapeDtypeStruct((2, _N, 16), _F32)],
    mesh=_mesh,
    scratch_types=[
        pltpu.VMEM((_NBLK,), jnp.int32),
        pltpu.VMEM((_NBLK, _HID), _F32),
        pltpu.VMEM((_NBLK, _HID), _F32),
        pltpu.VMEM((_EBLK, 16), _F32),
        pltpu.VMEM((_EBLK,), jnp.int32),
        pltpu.VMEM((_ZR, 16), _F32),
        pltpu.VMEM_SHARED((_N, 16), _F32),
        pltpu.SemaphoreType.DMA,
    ],
    compiler_params=_SC_PARAMS,
)(_embed_deg_body)


def _make_agg(nc):
  def body(*refs):
    srcr, dstr = refs[0], refs[1]
    hps = refs[2:2 + nc]
    so = refs[2 + nc]
    sb, db, rows, zbuf, accsh, sem = refs[3 + nc:]
    wid, cid, sid = _wid_cid_sid()
    _zero_fill(zbuf, _ZR)
    for c in range(nc):
      _zero_acc(accsh, zbuf, sid)
      plsc.subcore_barrier()

      def edge_body(j, _):
        base = wid * _EPT + j * _EBLK
        pltpu.sync_copy(srcr.at[pl.ds(base, _EBLK)], sb)
        pltpu.sync_copy(dstr.at[pl.ds(base, _EBLK)], db)
        pltpu.async_copy(hps[c].at[sb], rows, sem).wait()
        pltpu.sync_copy(rows, accsh.at[db], add=True)
        return 0
      lax.fori_loop(0, _NEB, edge_body, 0)
      plsc.subcore_barrier()
      _copy_out(accsh,
                lambda off, sz: so.at[c, cid, pl.ds(off, sz)], sid)
      plsc.subcore_barrier()

  return functools.partial(
      pl.kernel,
      out_type=jax.ShapeDtypeStruct((nc, 2, _N, 16), _F32),
      mesh=_mesh,
      scratch_types=[
          pltpu.VMEM((_EBLK,), jnp.int32),
          pltpu.VMEM((_EBLK,), jnp.int32),
          pltpu.VMEM((_EBLK, 16), _F32),
          pltpu.VMEM((_ZR, 16), _F32),
          pltpu.VMEM_SHARED((_N, 16), _F32),
          pltpu.SemaphoreType.DMA,
      ],
      compiler_params=_SC_PARAMS,
  )(body)


_agg3 = _make_agg(3)
_agg1 = _make_agg(1)


def _fuse_body(*refs):
  embs = refs[0:8]
  w = refs[8]
  outs = refs[9:]
  wv = w[...]
  for f in range(8):
    outs[f][...] = jnp.dot(embs[f][...], wv[_OFF[f]:_OFF[f] + _DS[f], :],
                           preferred_element_type=_F32)


_fuse_tables = pl.pallas_call(
    _fuse_body,
    out_shape=[jax.ShapeDtypeStruct((v, _HID), _F32) for v in _VS],
)


_NB = 4000
_GRID = _N // _NB


def _prep_body(dref, href, dvo, hp0, hp1, hp2):
  v = dref[...]
  deg = v[0] + v[1] + 2.0
  dv = lax.rsqrt(deg)
  dvo[...] = dv
  h = href[...]
  hpo = (hp0, hp1, hp2)
  for c in range(3):
    hpo[c][...] = dv * h[:, c * 16:(c + 1) * 16]


_prep = pl.pallas_call(
    _prep_body,
    grid=(_GRID,),
    in_specs=[
        pl.BlockSpec((2, _NB, 16), lambda i: (0, i, 0)),
        pl.BlockSpec((_NB, _HID), lambda i: (i, 0)),
    ],
    out_specs=[pl.BlockSpec((_NB, 16), lambda i: (i, 0))] * 4,
    out_shape=[jax.ShapeDtypeStruct((_N, 16), _F32)] * 4,
)


def _make_layer(nco):
  def body(sref, hp0r, hp1r, hp2r, dvr, wbr, br, *outs):
    sv = sref[...]
    dv = dvr[...]
    wb = wbr[...]
    bv = br[...]
    hps = (hp0r[...], hp1r[...], hp2r[...])
    acts = []
    for c in range(3):
      pre = dv * (sv[c, 0] + sv[c, 1] + 2.0 * hps[c]) + bv[c][None, :]
      acts.append(jnp.maximum(pre, 0.0))
    for co in range(nco):
      h = jnp.dot(acts[0], wb[0, co], preferred_element_type=_F32)
      h = h + jnp.dot(acts[1], wb[1, co], preferred_element_type=_F32)
      h = h + jnp.dot(acts[2], wb[2, co], preferred_element_type=_F32)
      outs[co][...] = dv * h

  return pl.pallas_call(
      body,
      grid=(_GRID,),
      in_specs=[
          pl.BlockSpec((3, 2, _NB, 16), lambda i: (0, 0, i, 0)),
          pl.BlockSpec((_NB, 16), lambda i: (i, 0)),
          pl.BlockSpec((_NB, 16), lambda i: (i, 0)),
          pl.BlockSpec((_NB, 16), lambda i: (i, 0)),
          pl.BlockSpec((_NB, 16), lambda i: (i, 0)),
          pl.BlockSpec((3, nco, 16, 16), lambda i: (0, 0, 0, 0)),
          pl.BlockSpec((3, 16), lambda i: (0, 0)),
      ],
      out_specs=[pl.BlockSpec((_NB, 16), lambda i: (i, 0))] * nco,
      out_shape=[jax.ShapeDtypeStruct((_N, 16), _F32)] * nco,
  )


_layer3 = _make_layer(3)
_layer1 = _make_layer(1)


def _final_body(sref, hpr, dvr, br, out):
  sv = sref[...]
  dv = dvr[...]
  out[...] = dv * (sv[0, 0] + sv[0, 1] + 2.0 * hpr[...]) + br[...][0][None, :]


_final = pl.pallas_call(
    _final_body,
    grid=(_GRID,),
    in_specs=[
        pl.BlockSpec((1, 2, _NB, 16), lambda i: (0, 0, i, 0)),
        pl.BlockSpec((_NB, 16), lambda i: (i, 0)),
        pl.BlockSpec((_NB, 16), lambda i: (i, 0)),
        pl.BlockSpec((1, 16), lambda i: (0, 0)),
    ],
    out_specs=pl.BlockSpec((_NB, 16), lambda i: (i, 0)),
    out_shape=jax.ShapeDtypeStruct((_N, 16), _F32),
)


def kernel(encode_andr_channel, encode_app_id, encode_device_model,
           encode_os_version, encode_dvce_manufacturer, encode_event_sub_type,
           collector_hour, collector_minute, emb_encode_andr_channel,
           emb_encode_app_id, emb_encode_device_model, emb_encode_os_version,
           emb_encode_dvce_manufacturer, emb_encode_event_sub_type,
           emb_collector_hour, emb_collector_minute, edge_index,
           W1, b1, W2, b2, W3, b3, W4, b4, W5, b5):
  idxs = [encode_andr_channel, encode_app_id, encode_device_model,
          encode_os_version, encode_dvce_manufacturer, encode_event_sub_type,
          collector_hour, collector_minute]
  embs = [emb_encode_andr_channel, emb_encode_app_id, emb_encode_device_model,
          emb_encode_os_version, emb_encode_dvce_manufacturer,
          emb_encode_event_sub_type, emb_collector_hour, emb_collector_minute]
  src = edge_index[0]
  dst = edge_index[1]

  tabs = _fuse_tables(*embs, W1)
  h1, deg2 = _embed_deg(*idxs, *tabs, dst)
  dv16, hp0, hp1, hp2 = _prep(deg2, h1)
  hp = (hp0, hp1, hp2)

  ws = [W2, W3, W4, W5]
  bs = [b1, b2, b3, b4]
  for i in range(4):
    s = _agg3(src, dst, *hp)
    nco = 3 if i < 3 else 1
    wb = ws[i].reshape(3, 16, nco, 16).transpose(0, 2, 1, 3)
    layer = _layer3 if i < 3 else _layer1
    hp = tuple(layer(s, *hp, dv16, wb, bs[i].reshape(3, 16)))

  s5 = _agg1(src, dst, hp[0])
  return _final(s5, hp[0], dv16, b5.reshape(1, 16))


# speedup vs baseline: 13.0224x; 1.4201x over previous
import functools

import jax
import jax.numpy as jnp
from jax import lax
from jax.experimental import pallas as pl
from jax.experimental.pallas import tpu as pltpu
from jax.experimental.pallas import tpu_sc as plsc

_F32 = jnp.float32
_N = 100000
_E = 1600000
_NTILES = 32
_EPT = _E // _NTILES
_EBLK = 400
_NEB = _EPT // _EBLK
_NPS = _N // 16
_ZR = 250
_VS = [100, 5000, 2000, 50, 200, 50, 24, 60]
_DS = [8, 8, 16, 10, 10, 8, 8, 8]
_OFF = [0, 8, 16, 32, 42, 52, 60, 68]
_HID = 48
_OUT = 16
_NBLK = 200
_NBLK_N = _N // _NBLK
_NB_FULL = 16

_mesh = plsc.VectorSubcoreMesh(
    core_axis_name="c", subcore_axis_name="s", num_cores=2, num_subcores=16)
_SC_PARAMS = pltpu.CompilerParams(use_tc_tiling_on_sc=False)


def _wid_cid_sid():
  cid = lax.axis_index("c")
  sid = lax.axis_index("s")
  return cid * 16 + sid, cid, sid


def _zero_fill(ref, rows):
  def body(r, _):
    ref[r, :] = jnp.zeros((16,), _F32)
    return 0
  lax.fori_loop(0, rows, body, 0)


def _zero_acc(accsh, zbuf, sid):
  def body(r, _):
    pltpu.sync_copy(zbuf, accsh.at[pl.ds(sid * _NPS + r * _ZR, _ZR)])
    return 0
  lax.fori_loop(0, 25, body, 0)


def _copy_out(accsh, dst_at, sid):
  @pl.when(sid < 15)
  def _():
    off = pl.multiple_of(sid * 6256, 8)
    pltpu.sync_copy(accsh.at[pl.ds(off, 6256)], dst_at(off, 6256))

  @pl.when(sid == 15)
  def _():
    pltpu.sync_copy(accsh.at[pl.ds(93840, 6160)], dst_at(93840, 6160))


def _embed_deg_body(*refs):
  idxs = refs[0:8]
  tabs = refs[8:16]
  dstr = refs[16]
  h1o = refs[17]
  dego = refs[18]
  ib, gbuf, acc, ones, eb, zbuf, accsh, sem = refs[19:]
  wid, cid, sid = _wid_cid_sid()

  _zero_fill(zbuf, _ZR)
  def ones_body(r, _):
    ones[r, :] = jnp.full((16,), 1.0, _F32)
    return 0
  lax.fori_loop(0, _EBLK, ones_body, 0)

  _zero_acc(accsh, zbuf, sid)
  plsc.subcore_barrier()

  def deg_body(j, _):
    base = wid * _EPT + j * _EBLK
    pltpu.sync_copy(dstr.at[pl.ds(base, _EBLK)], eb)
    pltpu.sync_copy(ones, accsh.at[eb], add=True)
    return 0
  lax.fori_loop(0, _NEB, deg_body, 0)
  plsc.subcore_barrier()
  _copy_out(accsh, lambda off, sz: dego.at[cid, pl.ds(off, sz)], sid)

  nblk = jnp.where(wid < (_NBLK_N - (_NB_FULL - 1) * _NTILES),
                   _NB_FULL, _NB_FULL - 1)

  def emb_body(b, _):
    base = (wid + _NTILES * b) * _NBLK
    pltpu.sync_copy(idxs[0].at[pl.ds(base, _NBLK)], ib)
    pltpu.async_copy(tabs[0].at[ib], acc, sem).wait()
    for f in range(1, 8):
      pltpu.sync_copy(idxs[f].at[pl.ds(base, _NBLK)], ib)
      pltpu.async_copy(tabs[f].at[ib], gbuf, sem).wait()
      def add_body(r, _):
        for c in range(3):
          plsc.addupdate(acc.at[r, pl.ds(c * 16, 16)],
                         gbuf[r, pl.ds(c * 16, 16)])
        return 0
      lax.fori_loop(0, _NBLK, add_body, 0)
    pltpu.sync_copy(acc, h1o.at[pl.ds(base, _NBLK)])
    return 0
  lax.fori_loop(0, nblk, emb_body, 0)


_embed_deg = functools.partial(
    pl.kernel,
    out_type=[jax.ShapeDtypeStruct((_N, _HID), _F32),
              jax.ShapeDtypeStruct((2, _N, 16), _F32)],
    mesh=_mesh,
    scratch_types=[
        pltpu.VMEM((_NBLK,), jnp.int32),
        pltpu.VMEM((_NBLK, _HID), _F32),
        pltpu.VMEM((_NBLK, _HID), _F32),
        pltpu.VMEM((_EBLK, 16), _F32),
        pltpu.VMEM((_EBLK,), jnp.int32),
        pltpu.VMEM((_ZR, 16), _F32),
        pltpu.VMEM_SHARED((_N, 16), _F32),
        pltpu.SemaphoreType.DMA,
    ],
    compiler_params=_SC_PARAMS,
)(_embed_deg_body)


def _make_agg(nc):
  def body(*refs):
    srcr, dstr = refs[0], refs[1]
    hps = refs[2:2 + nc]
    so = refs[2 + nc]
    sb, db, rows, zbuf, accsh, sg, ss, sis, sid_s = refs[3 + nc:]
    wid, cid, sid = _wid_cid_sid()
    _zero_fill(zbuf, _ZR)
    ebase = wid * _EPT
    for c in range(nc):
      _zero_acc(accsh, zbuf, sid)
      plsc.subcore_barrier()

      pltpu.sync_copy(srcr.at[pl.ds(ebase, _EBLK)], sb.at[0])
      pltpu.sync_copy(dstr.at[pl.ds(ebase, _EBLK)], db.at[0])
      pltpu.async_copy(hps[c].at[sb.at[0]], rows.at[0], sg)

      def edge_body(j, _):
        m = lax.rem(j, 2)
        nm = lax.rem(j + 1, 2)
        nb = ebase + (j + 1) * _EBLK

        @pl.when(j + 1 < _NEB)
        def _():
          pltpu.async_copy(srcr.at[pl.ds(nb, _EBLK)], sb.at[nm], sis)
          pltpu.async_copy(dstr.at[pl.ds(nb, _EBLK)], db.at[nm], sid_s)

        pltpu.make_async_copy(hps[c].at[sb.at[m]], rows.at[m], sg).wait()
        pltpu.async_copy(rows.at[m], accsh.at[db.at[m]], ss, add=True)

        @pl.when(j + 1 < _NEB)
        def _():
          pltpu.make_async_copy(srcr.at[pl.ds(nb, _EBLK)], sb.at[nm],
                                sis).wait()
          pltpu.make_async_copy(dstr.at[pl.ds(nb, _EBLK)], db.at[nm],
                                sid_s).wait()
          pltpu.async_copy(hps[c].at[sb.at[nm]], rows.at[nm], sg)

        pltpu.make_async_copy(rows.at[m], accsh.at[db.at[m]], ss).wait()
        return 0
      lax.fori_loop(0, _NEB, edge_body, 0)
      plsc.subcore_barrier()
      _copy_out(accsh,
                lambda off, sz: so.at[c, cid, pl.ds(off, sz)], sid)
      plsc.subcore_barrier()

  return functools.partial(
      pl.kernel,
      out_type=jax.ShapeDtypeStruct((nc, 2, _N, 16), _F32),
      mesh=_mesh,
      scratch_types=[
          pltpu.VMEM((2, _EBLK), jnp.int32),
          pltpu.VMEM((2, _EBLK), jnp.int32),
          pltpu.VMEM((2, _EBLK, 16), _F32),
          pltpu.VMEM((_ZR, 16), _F32),
          pltpu.VMEM_SHARED((_N, 16), _F32),
          pltpu.SemaphoreType.DMA,
          pltpu.SemaphoreType.DMA,
          pltpu.SemaphoreType.DMA,
          pltpu.SemaphoreType.DMA,
      ],
      compiler_params=_SC_PARAMS,
  )(body)


_agg3 = _make_agg(3)
_agg1 = _make_agg(1)


def _fuse_body(*refs):
  embs = refs[0:8]
  w = refs[8]
  outs = refs[9:]
  wv = w[...]
  for f in range(8):
    outs[f][...] = jnp.dot(embs[f][...], wv[_OFF[f]:_OFF[f] + _DS[f], :],
                           preferred_element_type=_F32)


_fuse_tables = pl.pallas_call(
    _fuse_body,
    out_shape=[jax.ShapeDtypeStruct((v, _HID), _F32) for v in _VS],
)


_NB = 4000
_GRID = _N // _NB


def _prep_body(dref, href, dvo, hp0, hp1, hp2):
  v = dref[...]
  deg = v[0] + v[1] + 2.0
  dv = lax.rsqrt(deg)
  dvo[...] = dv
  h = href[...]
  hpo = (hp0, hp1, hp2)
  for c in range(3):
    hpo[c][...] = dv * h[:, c * 16:(c + 1) * 16]


_prep = pl.pallas_call(
    _prep_body,
    grid=(_GRID,),
    in_specs=[
        pl.BlockSpec((2, _NB, 16), lambda i: (0, i, 0)),
        pl.BlockSpec((_NB, _HID), lambda i: (i, 0)),
    ],
    out_specs=[pl.BlockSpec((_NB, 16), lambda i: (i, 0))] * 4,
    out_shape=[jax.ShapeDtypeStruct((_N, 16), _F32)] * 4,
)


def _make_layer(nco):
  def body(sref, hp0r, hp1r, hp2r, dvr, wbr, br, *outs):
    sv = sref[...]
    dv = dvr[...]
    wb = wbr[...]
    bv = br[...]
    hps = (hp0r[...], hp1r[...], hp2r[...])
    acts = []
    for c in range(3):
      pre = dv * (sv[c, 0] + sv[c, 1] + 2.0 * hps[c]) + bv[c][None, :]
      acts.append(jnp.maximum(pre, 0.0))
    for co in range(nco):
      h = jnp.dot(acts[0], wb[0, co], preferred_element_type=_F32)
      h = h + jnp.dot(acts[1], wb[1, co], preferred_element_type=_F32)
      h = h + jnp.dot(acts[2], wb[2, co], preferred_element_type=_F32)
      outs[co][...] = dv * h

  return pl.pallas_call(
      body,
      grid=(_GRID,),
      in_specs=[
          pl.BlockSpec((3, 2, _NB, 16), lambda i: (0, 0, i, 0)),
          pl.BlockSpec((_NB, 16), lambda i: (i, 0)),
          pl.BlockSpec((_NB, 16), lambda i: (i, 0)),
          pl.BlockSpec((_NB, 16), lambda i: (i, 0)),
          pl.BlockSpec((_NB, 16), lambda i: (i, 0)),
          pl.BlockSpec((3, nco, 16, 16), lambda i: (0, 0, 0, 0)),
          pl.BlockSpec((3, 16), lambda i: (0, 0)),
      ],
      out_specs=[pl.BlockSpec((_NB, 16), lambda i: (i, 0))] * nco,
      out_shape=[jax.ShapeDtypeStruct((_N, 16), _F32)] * nco,
  )


_layer3 = _make_layer(3)
_layer1 = _make_layer(1)


def _final_body(sref, hpr, dvr, br, out):
  sv = sref[...]
  dv = dvr[...]
  out[...] = dv * (sv[0, 0] + sv[0, 1] + 2.0 * hpr[...]) + br[...][0][None, :]


_final = pl.pallas_call(
    _final_body,
    grid=(_GRID,),
    in_specs=[
        pl.BlockSpec((1, 2, _NB, 16), lambda i: (0, 0, i, 0)),
        pl.BlockSpec((_NB, 16), lambda i: (i, 0)),
        pl.BlockSpec((_NB, 16), lambda i: (i, 0)),
        pl.BlockSpec((1, 16), lambda i: (0, 0)),
    ],
    out_specs=pl.BlockSpec((_NB, 16), lambda i: (i, 0)),
    out_shape=jax.ShapeDtypeStruct((_N, 16), _F32),
)


def kernel(encode_andr_channel, encode_app_id, encode_device_model,
           encode_os_version, encode_dvce_manufacturer, encode_event_sub_type,
           collector_hour, collector_minute, emb_encode_andr_channel,
           emb_encode_app_id, emb_encode_device_model, emb_encode_os_version,
           emb_encode_dvce_manufacturer, emb_encode_event_sub_type,
           emb_collector_hour, emb_collector_minute, edge_index,
           W1, b1, W2, b2, W3, b3, W4, b4, W5, b5):
  idxs = [encode_andr_channel, encode_app_id, encode_device_model,
          encode_os_version, encode_dvce_manufacturer, encode_event_sub_type,
          collector_hour, collector_minute]
  embs = [emb_encode_andr_channel, emb_encode_app_id, emb_encode_device_model,
          emb_encode_os_version, emb_encode_dvce_manufacturer,
          emb_encode_event_sub_type, emb_collector_hour, emb_collector_minute]
  src = edge_index[0]
  dst = edge_index[1]

  tabs = _fuse_tables(*embs, W1)
  h1, deg2 = _embed_deg(*idxs, *tabs, dst)
  dv16, hp0, hp1, hp2 = _prep(deg2, h1)
  hp = (hp0, hp1, hp2)

  ws = [W2, W3, W4, W5]
  bs = [b1, b2, b3, b4]
  for i in range(4):
    s = _agg3(src, dst, *hp)
    nco = 3 if i < 3 else 1
    wb = ws[i].reshape(3, 16, nco, 16).transpose(0, 2, 1, 3)
    layer = _layer3 if i < 3 else _layer1
    hp = tuple(layer(s, *hp, dv16, wb, bs[i].reshape(3, 16)))

  s5 = _agg1(src, dst, hp[0])
  return _final(s5, hp[0], dv16, b5.reshape(1, 16))


# speedup vs baseline: 15.3890x; 1.1817x over previous
import functools

import jax
import jax.numpy as jnp
from jax import lax
from jax.experimental import pallas as pl
from jax.experimental.pallas import tpu as pltpu
from jax.experimental.pallas import tpu_sc as plsc

_F32 = jnp.float32
_N = 100000
_E = 1600000
_NTILES = 32
_EPT = _E // _NTILES
_EBLK = 400
_NEB = _EPT // _EBLK
_NPS = _N // 16
_ZR = 250
_VS = [100, 5000, 2000, 50, 200, 50, 24, 60]
_DS = [8, 8, 16, 10, 10, 8, 8, 8]
_OFF = [0, 8, 16, 32, 42, 52, 60, 68]
_HID = 48
_OUT = 16
_NBLK = 200
_NBLK_N = _N // _NBLK
_NB_FULL = 16

_mesh = plsc.VectorSubcoreMesh(
    core_axis_name="c", subcore_axis_name="s", num_cores=2, num_subcores=16)
_SC_PARAMS = pltpu.CompilerParams(use_tc_tiling_on_sc=False)


def _wid_cid_sid():
  cid = lax.axis_index("c")
  sid = lax.axis_index("s")
  return cid * 16 + sid, cid, sid


def _zero_fill(ref, rows):
  def body(r, _):
    ref[r, :] = jnp.zeros((16,), _F32)
    return 0
  lax.fori_loop(0, rows, body, 0)


def _zero_acc(accsh, zbuf, sid):
  def body(r, _):
    pltpu.sync_copy(zbuf, accsh.at[pl.ds(sid * _NPS + r * _ZR, _ZR)])
    return 0
  lax.fori_loop(0, 25, body, 0)


def _copy_out(accsh, dst_at, sid):
  @pl.when(sid < 15)
  def _():
    off = pl.multiple_of(sid * 6256, 8)
    pltpu.sync_copy(accsh.at[pl.ds(off, 6256)], dst_at(off, 6256))

  @pl.when(sid == 15)
  def _():
    pltpu.sync_copy(accsh.at[pl.ds(93840, 6160)], dst_at(93840, 6160))


def _embed_deg_body(*refs):
  idxs = refs[0:8]
  tabs = refs[8:16]
  dstr = refs[16]
  h1o = refs[17]
  dego = refs[18]
  ib, gbuf, acc, ones, eb, zbuf, accsh, sem = refs[19:]
  wid, cid, sid = _wid_cid_sid()

  _zero_fill(zbuf, _ZR)
  def ones_body(r, _):
    ones[r, :] = jnp.full((16,), 1.0, _F32)
    return 0
  lax.fori_loop(0, _EBLK, ones_body, 0)

  _zero_acc(accsh, zbuf, sid)
  plsc.subcore_barrier()

  def deg_body(j, _):
    base = wid * _EPT + j * _EBLK
    pltpu.sync_copy(dstr.at[pl.ds(base, _EBLK)], eb)
    pltpu.sync_copy(ones, accsh.at[eb], add=True)
    return 0
  lax.fori_loop(0, _NEB, deg_body, 0)
  plsc.subcore_barrier()
  _copy_out(accsh, lambda off, sz: dego.at[cid, pl.ds(off, sz)], sid)

  nblk = jnp.where(wid < (_NBLK_N - (_NB_FULL - 1) * _NTILES),
                   _NB_FULL, _NB_FULL - 1)

  def emb_body(b, _):
    base = (wid + _NTILES * b) * _NBLK
    pltpu.sync_copy(idxs[0].at[pl.ds(base, _NBLK)], ib)
    pltpu.async_copy(tabs[0].at[ib], acc, sem).wait()
    for f in range(1, 8):
      pltpu.sync_copy(idxs[f].at[pl.ds(base, _NBLK)], ib)
      pltpu.async_copy(tabs[f].at[ib], gbuf, sem).wait()
      def add_body(r, _):
        for c in range(3):
          plsc.addupdate(acc.at[r, pl.ds(c * 16, 16)],
                         gbuf[r, pl.ds(c * 16, 16)])
        return 0
      lax.fori_loop(0, _NBLK, add_body, 0)
    pltpu.sync_copy(acc, h1o.at[pl.ds(base, _NBLK)])
    return 0
  lax.fori_loop(0, nblk, emb_body, 0)


_embed_deg = functools.partial(
    pl.kernel,
    out_type=[jax.ShapeDtypeStruct((_N, _HID), _F32),
              jax.ShapeDtypeStruct((2, _N, 16), _F32)],
    mesh=_mesh,
    scratch_types=[
        pltpu.VMEM((_NBLK,), jnp.int32),
        pltpu.VMEM((_NBLK, _HID), _F32),
        pltpu.VMEM((_NBLK, _HID), _F32),
        pltpu.VMEM((_EBLK, 16), _F32),
        pltpu.VMEM((_EBLK,), jnp.int32),
        pltpu.VMEM((_ZR, 16), _F32),
        pltpu.VMEM_SHARED((_N, 16), _F32),
        pltpu.SemaphoreType.DMA,
    ],
    compiler_params=_SC_PARAMS,
)(_embed_deg_body)


def _make_agg(nc):
  def body(*refs):
    eir = refs[0]
    hps = refs[1:1 + nc]
    so = refs[1 + nc]
    eib, rows, zbuf, accsh, sg, ss, si = refs[2 + nc:]
    wid, cid, sid = _wid_cid_sid()
    _zero_fill(zbuf, _ZR)
    ebase = wid * _EPT

    def ei_slice(j):
      return eir.at[:, pl.ds(ebase + j * _EBLK, _EBLK)]

    def gather(c, j, slot3, slot4):
      return pltpu.make_async_copy(hps[c].at[eib.at[slot4, 0]],
                                   rows.at[slot3], sg.at[slot3])

    for c in range(nc):
      _zero_acc(accsh, zbuf, sid)
      plsc.subcore_barrier()

      pltpu.sync_copy(ei_slice(0), eib.at[0])
      gather(c, 0, 0, 0).start()
      pltpu.sync_copy(ei_slice(1), eib.at[1])
      gather(c, 1, 1, 1).start()
      pltpu.async_copy(ei_slice(2), eib.at[2], si)

      def edge_body(j, _):
        m3 = lax.rem(j, 3)
        m4 = lax.rem(j, 4)
        gather(c, j, m3, m4).wait()
        pltpu.async_copy(rows.at[m3], accsh.at[eib.at[m4, 1]], ss, add=True)

        @pl.when(j + 2 < _NEB)
        def _():
          n3 = lax.rem(j + 2, 3)
          n4 = lax.rem(j + 2, 4)
          pltpu.make_async_copy(ei_slice(j + 2), eib.at[n4], si).wait()
          gather(c, j + 2, n3, n4).start()

        @pl.when(j + 3 < _NEB)
        def _():
          pltpu.async_copy(ei_slice(j + 3), eib.at[lax.rem(j + 3, 4)], si)

        pltpu.make_async_copy(rows.at[m3], accsh.at[eib.at[m4, 1]],
                              ss).wait()
        return 0
      lax.fori_loop(0, _NEB, edge_body, 0)
      plsc.subcore_barrier()
      _copy_out(accsh,
                lambda off, sz: so.at[c, cid, pl.ds(off, sz)], sid)
      plsc.subcore_barrier()

  return functools.partial(
      pl.kernel,
      out_type=jax.ShapeDtypeStruct((nc, 2, _N, 16), _F32),
      mesh=_mesh,
      scratch_types=[
          pltpu.VMEM((4, 2, _EBLK), jnp.int32),
          pltpu.VMEM((3, _EBLK, 16), _F32),
          pltpu.VMEM((_ZR, 16), _F32),
          pltpu.VMEM_SHARED((_N, 16), _F32),
          pltpu.SemaphoreType.DMA((3,)),
          pltpu.SemaphoreType.DMA,
          pltpu.SemaphoreType.DMA,
      ],
      compiler_params=_SC_PARAMS,
  )(body)


_agg3 = _make_agg(3)
_agg1 = _make_agg(1)


def _fuse_body(*refs):
  embs = refs[0:8]
  w = refs[8]
  outs = refs[9:]
  wv = w[...]
  for f in range(8):
    outs[f][...] = jnp.dot(embs[f][...], wv[_OFF[f]:_OFF[f] + _DS[f], :],
                           preferred_element_type=_F32)


_fuse_tables = pl.pallas_call(
    _fuse_body,
    out_shape=[jax.ShapeDtypeStruct((v, _HID), _F32) for v in _VS],
)


_NB = 4000
_GRID = _N // _NB


def _prep_body(dref, href, dvo, hp0, hp1, hp2):
  v = dref[...]
  deg = v[0] + v[1] + 2.0
  dv = lax.rsqrt(deg)
  dvo[...] = dv
  h = href[...]
  hpo = (hp0, hp1, hp2)
  for c in range(3):
    hpo[c][...] = dv * h[:, c * 16:(c + 1) * 16]


_prep = pl.pallas_call(
    _prep_body,
    grid=(_GRID,),
    in_specs=[
        pl.BlockSpec((2, _NB, 16), lambda i: (0, i, 0)),
        pl.BlockSpec((_NB, _HID), lambda i: (i, 0)),
    ],
    out_specs=[pl.BlockSpec((_NB, 16), lambda i: (i, 0))] * 4,
    out_shape=[jax.ShapeDtypeStruct((_N, 16), _F32)] * 4,
)


def _make_layer(nco):
  def body(sref, hp0r, hp1r, hp2r, dvr, wbr, br, *outs):
    sv = sref[...]
    dv = dvr[...]
    wb = wbr[...]
    bv = br[...]
    hps = (hp0r[...], hp1r[...], hp2r[...])
    acts = []
    for c in range(3):
      pre = dv * (sv[c, 0] + sv[c, 1] + 2.0 * hps[c]) + bv[c][None, :]
      acts.append(jnp.maximum(pre, 0.0))
    for co in range(nco):
      h = jnp.dot(acts[0], wb[0, co], preferred_element_type=_F32)
      h = h + jnp.dot(acts[1], wb[1, co], preferred_element_type=_F32)
      h = h + jnp.dot(acts[2], wb[2, co], preferred_element_type=_F32)
      outs[co][...] = dv * h

  return pl.pallas_call(
      body,
      grid=(_GRID,),
      in_specs=[
          pl.BlockSpec((3, 2, _NB, 16), lambda i: (0, 0, i, 0)),
          pl.BlockSpec((_NB, 16), lambda i: (i, 0)),
          pl.BlockSpec((_NB, 16), lambda i: (i, 0)),
          pl.BlockSpec((_NB, 16), lambda i: (i, 0)),
          pl.BlockSpec((_NB, 16), lambda i: (i, 0)),
          pl.BlockSpec((3, nco, 16, 16), lambda i: (0, 0, 0, 0)),
          pl.BlockSpec((3, 16), lambda i: (0, 0)),
      ],
      out_specs=[pl.BlockSpec((_NB, 16), lambda i: (i, 0))] * nco,
      out_shape=[jax.ShapeDtypeStruct((_N, 16), _F32)] * nco,
  )


_layer3 = _make_layer(3)
_layer1 = _make_layer(1)


def _final_body(sref, hpr, dvr, br, out):
  sv = sref[...]
  dv = dvr[...]
  out[...] = dv * (sv[0, 0] + sv[0, 1] + 2.0 * hpr[...]) + br[...][0][None, :]


_final = pl.pallas_call(
    _final_body,
    grid=(_GRID,),
    in_specs=[
        pl.BlockSpec((1, 2, _NB, 16), lambda i: (0, 0, i, 0)),
        pl.BlockSpec((_NB, 16), lambda i: (i, 0)),
        pl.BlockSpec((_NB, 16), lambda i: (i, 0)),
        pl.BlockSpec((1, 16), lambda i: (0, 0)),
    ],
    out_specs=pl.BlockSpec((_NB, 16), lambda i: (i, 0)),
    out_shape=jax.ShapeDtypeStruct((_N, 16), _F32),
)


def kernel(encode_andr_channel, encode_app_id, encode_device_model,
           encode_os_version, encode_dvce_manufacturer, encode_event_sub_type,
           collector_hour, collector_minute, emb_encode_andr_channel,
           emb_encode_app_id, emb_encode_device_model, emb_encode_os_version,
           emb_encode_dvce_manufacturer, emb_encode_event_sub_type,
           emb_collector_hour, emb_collector_minute, edge_index,
           W1, b1, W2, b2, W3, b3, W4, b4, W5, b5):
  idxs = [encode_andr_channel, encode_app_id, encode_device_model,
          encode_os_version, encode_dvce_manufacturer, encode_event_sub_type,
          collector_hour, collector_minute]
  embs = [emb_encode_andr_channel, emb_encode_app_id, emb_encode_device_model,
          emb_encode_os_version, emb_encode_dvce_manufacturer,
          emb_encode_event_sub_type, emb_collector_hour, emb_collector_minute]
  dst = edge_index[1]

  tabs = _fuse_tables(*embs, W1)
  h1, deg2 = _embed_deg(*idxs, *tabs, dst)
  dv16, hp0, hp1, hp2 = _prep(deg2, h1)
  hp = (hp0, hp1, hp2)

  ws = [W2, W3, W4, W5]
  bs = [b1, b2, b3, b4]
  for i in range(4):
    s = _agg3(edge_index, *hp)
    nco = 3 if i < 3 else 1
    wb = ws[i].reshape(3, 16, nco, 16).transpose(0, 2, 1, 3)
    layer = _layer3 if i < 3 else _layer1
    hp = tuple(layer(s, *hp, dv16, wb, bs[i].reshape(3, 16)))

  s5 = _agg1(edge_index, hp[0])
  return _final(s5, hp[0], dv16, b5.reshape(1, 16))


# speedup vs baseline: 28.4907x; 1.8514x over previous
import functools

import jax
import jax.numpy as jnp
from jax import lax
from jax.experimental import pallas as pl
from jax.experimental.pallas import tpu as pltpu
from jax.experimental.pallas import tpu_sc as plsc

_F32 = jnp.float32
_N = 100000
_E = 1600000
_NTILES = 32
_EPT = _E // _NTILES
_EBLK = 400
_NEB = _EPT // _EBLK
_NPS = _N // 16
_ZR = 250
_VS = [100, 5000, 2000, 50, 200, 50, 24, 60]
_DS = [8, 8, 16, 10, 10, 8, 8, 8]
_OFF = [0, 8, 16, 32, 42, 52, 60, 68]
_HID = 48
_OUT = 16
_NBLK = 200
_NBLK_N = _N // _NBLK
_NB_FULL = 16

_mesh = plsc.VectorSubcoreMesh(
    core_axis_name="c", subcore_axis_name="s", num_cores=2, num_subcores=16)
_SC_PARAMS = pltpu.CompilerParams(use_tc_tiling_on_sc=False)


def _wid_cid_sid():
  cid = lax.axis_index("c")
  sid = lax.axis_index("s")
  return cid * 16 + sid, cid, sid


def _zero_fill(ref, rows):
  def body(r, _):
    ref[r, :] = jnp.zeros((16,), _F32)
    return 0
  lax.fori_loop(0, rows, body, 0)


def _zero_acc(accsh, zbuf, sid):
  def body(r, _):
    pltpu.sync_copy(zbuf, accsh.at[pl.ds(sid * _NPS + r * _ZR, _ZR)])
    return 0
  lax.fori_loop(0, 25, body, 0)


def _copy_out(accsh, dst_at, sid):
  @pl.when(sid < 15)
  def _():
    off = pl.multiple_of(sid * 6256, 8)
    pltpu.sync_copy(accsh.at[pl.ds(off, 6256)], dst_at(off, 6256))

  @pl.when(sid == 15)
  def _():
    pltpu.sync_copy(accsh.at[pl.ds(93840, 6160)], dst_at(93840, 6160))


def _embed_deg_body(*refs):
  idxs = refs[0:8]
  tabs = refs[8:32]
  dstr = refs[32]
  h1o = refs[33:36]
  dego = refs[36]
  ib, gb0, gb1, gb2, ac0, ac1, ac2, ones, eb, zbuf, accsh, sem = refs[37:]
  gbs = (gb0, gb1, gb2)
  acs = (ac0, ac1, ac2)
  wid, cid, sid = _wid_cid_sid()

  _zero_fill(zbuf, _ZR)
  def ones_body(r, _):
    ones[r, :] = jnp.full((16,), 1.0, _F32)
    return 0
  lax.fori_loop(0, _EBLK, ones_body, 0)

  _zero_acc(accsh, zbuf, sid)
  plsc.subcore_barrier()

  def deg_body(j, _):
    base = wid * _EPT + j * _EBLK
    pltpu.sync_copy(dstr.at[pl.ds(base, _EBLK)], eb)
    pltpu.sync_copy(ones, accsh.at[eb], add=True)
    return 0
  lax.fori_loop(0, _NEB, deg_body, 0)
  plsc.subcore_barrier()
  _copy_out(accsh, lambda off, sz: dego.at[cid, pl.ds(off, sz)], sid)

  nblk = jnp.where(wid < (_NBLK_N - (_NB_FULL - 1) * _NTILES),
                   _NB_FULL, _NB_FULL - 1)

  def emb_body(b, _):
    base = (wid + _NTILES * b) * _NBLK
    pltpu.sync_copy(idxs[0].at[pl.ds(base, _NBLK)], ib)
    for c in range(3):
      pltpu.async_copy(tabs[c].at[ib], acs[c], sem).wait()
    for f in range(1, 8):
      pltpu.sync_copy(idxs[f].at[pl.ds(base, _NBLK)], ib)
      for c in range(3):
        pltpu.async_copy(tabs[f * 3 + c].at[ib], gbs[c], sem).wait()
      def add_body(r, _):
        for c in range(3):
          plsc.addupdate(acs[c].at[r], gbs[c][r])
        return 0
      lax.fori_loop(0, _NBLK, add_body, 0)
    for c in range(3):
      pltpu.sync_copy(acs[c], h1o[c].at[pl.ds(base, _NBLK)])
    return 0
  lax.fori_loop(0, nblk, emb_body, 0)


_embed_deg = functools.partial(
    pl.kernel,
    out_type=[jax.ShapeDtypeStruct((102400, 16), _F32)] * 3
    + [jax.ShapeDtypeStruct((2, 102400, 16), _F32)],
    mesh=_mesh,
    scratch_types=[
        pltpu.VMEM((_NBLK,), jnp.int32),
        pltpu.VMEM((_NBLK, 16), _F32),
        pltpu.VMEM((_NBLK, 16), _F32),
        pltpu.VMEM((_NBLK, 16), _F32),
        pltpu.VMEM((_NBLK, 16), _F32),
        pltpu.VMEM((_NBLK, 16), _F32),
        pltpu.VMEM((_NBLK, 16), _F32),
        pltpu.VMEM((_EBLK, 16), _F32),
        pltpu.VMEM((_EBLK,), jnp.int32),
        pltpu.VMEM((_ZR, 16), _F32),
        pltpu.VMEM_SHARED((_N, 16), _F32),
        pltpu.SemaphoreType.DMA,
    ],
    compiler_params=_SC_PARAMS,
)(_embed_deg_body)


def _make_agg(nc):
  def body(*refs):
    eir = refs[0]
    hps = refs[1:1 + nc]
    so = refs[1 + nc]
    eib, rows, zbuf, accsh, sg, ss, si = refs[2 + nc:]
    wid, cid, sid = _wid_cid_sid()
    _zero_fill(zbuf, _ZR)
    ebase = wid * _EPT

    def ei_slice(j):
      return eir.at[:, pl.ds(ebase + j * _EBLK, _EBLK)]

    def gather(c, j, slot3, slot4):
      return pltpu.make_async_copy(hps[c].at[eib.at[slot4, 0]],
                                   rows.at[slot3], sg.at[slot3])

    for c in range(nc):
      _zero_acc(accsh, zbuf, sid)
      plsc.subcore_barrier()

      pltpu.sync_copy(ei_slice(0), eib.at[0])
      gather(c, 0, 0, 0).start()
      pltpu.sync_copy(ei_slice(1), eib.at[1])
      gather(c, 1, 1, 1).start()
      pltpu.async_copy(ei_slice(2), eib.at[2], si)

      def edge_body(j, _):
        m3 = lax.rem(j, 3)
        m4 = lax.rem(j, 4)
        gather(c, j, m3, m4).wait()
        pltpu.async_copy(rows.at[m3], accsh.at[eib.at[m4, 1]], ss, add=True)

        @pl.when(j + 2 < _NEB)
        def _():
          n3 = lax.rem(j + 2, 3)
          n4 = lax.rem(j + 2, 4)
          pltpu.make_async_copy(ei_slice(j + 2), eib.at[n4], si).wait()
          gather(c, j + 2, n3, n4).start()

        @pl.when(j + 3 < _NEB)
        def _():
          pltpu.async_copy(ei_slice(j + 3), eib.at[lax.rem(j + 3, 4)], si)

        pltpu.make_async_copy(rows.at[m3], accsh.at[eib.at[m4, 1]],
                              ss).wait()
        return 0
      lax.fori_loop(0, _NEB, edge_body, 0)
      plsc.subcore_barrier()
      _copy_out(accsh,
                lambda off, sz: so.at[c, cid, pl.ds(off, sz)], sid)
      plsc.subcore_barrier()

  return functools.partial(
      pl.kernel,
      out_type=jax.ShapeDtypeStruct((nc, 2, 102400, 16), _F32),
      mesh=_mesh,
      scratch_types=[
          pltpu.VMEM((4, 2, _EBLK), jnp.int32),
          pltpu.VMEM((3, _EBLK, 16), _F32),
          pltpu.VMEM((_ZR, 16), _F32),
          pltpu.VMEM_SHARED((_N, 16), _F32),
          pltpu.SemaphoreType.DMA((3,)),
          pltpu.SemaphoreType.DMA,
          pltpu.SemaphoreType.DMA,
      ],
      compiler_params=_SC_PARAMS,
  )(body)


_agg3 = _make_agg(3)
_agg1 = _make_agg(1)


def _fuse_body(*refs):
  embs = refs[0:8]
  w = refs[8]
  outs = refs[9:]
  wv = w[...]
  for f in range(8):
    ev = embs[f][...]
    for c in range(3):
      outs[f * 3 + c][...] = jnp.dot(
          ev, wv[_OFF[f]:_OFF[f] + _DS[f], c * 16:(c + 1) * 16],
          preferred_element_type=_F32)


_fuse_tables = pl.pallas_call(
    _fuse_body,
    out_shape=[jax.ShapeDtypeStruct((v, 16), _F32)
               for v in _VS for _ in range(3)],
)


_NPAD = 102400
_NPK = _NPAD // 8
_NBP = 1280
_GRID = _NPK // _NBP


def _prep_body(dref, h0r, h1r, h2r, dvo, hp0, hp1, hp2):
  v = dref[...]
  dv = lax.rsqrt(v[0] + v[1] + 2.0)
  dvo[...] = dv
  hpo = (hp0, hp1, hp2)
  for c, hr in enumerate((h0r, h1r, h2r)):
    hpo[c][...] = dv * hr[...]


_prep = pl.pallas_call(
    _prep_body,
    grid=(_GRID,),
    in_specs=[pl.BlockSpec((2, _NBP, 128), lambda i: (0, i, 0))]
    + [pl.BlockSpec((_NBP, 128), lambda i: (i, 0))] * 3,
    out_specs=[pl.BlockSpec((_NBP, 128), lambda i: (i, 0))] * 4,
    out_shape=[jax.ShapeDtypeStruct((_NPK, 128), _F32)] * 4,
)


def _make_layer(nco):
  def body(sref, hp0r, hp1r, hp2r, dvr, wbr, br, *outs):
    sv = sref[...]
    dv = dvr[...]
    wb = wbr[...]
    bv = br[...]
    hps = (hp0r[...], hp1r[...], hp2r[...])
    acts = []
    for c in range(3):
      pre = dv * (sv[c, 0] + sv[c, 1] + 2.0 * hps[c]) + bv[c][None, :]
      acts.append(jnp.maximum(pre, 0.0))
    for co in range(nco):
      h = jnp.dot(acts[0], wb[0, co], preferred_element_type=_F32)
      h = h + jnp.dot(acts[1], wb[1, co], preferred_element_type=_F32)
      h = h + jnp.dot(acts[2], wb[2, co], preferred_element_type=_F32)
      outs[co][...] = dv * h

  return pl.pallas_call(
      body,
      grid=(_GRID,),
      in_specs=[
          pl.BlockSpec((3, 2, _NBP, 128), lambda i: (0, 0, i, 0)),
          pl.BlockSpec((_NBP, 128), lambda i: (i, 0)),
          pl.BlockSpec((_NBP, 128), lambda i: (i, 0)),
          pl.BlockSpec((_NBP, 128), lambda i: (i, 0)),
          pl.BlockSpec((_NBP, 128), lambda i: (i, 0)),
          pl.BlockSpec((3, nco, 128, 128), lambda i: (0, 0, 0, 0)),
          pl.BlockSpec((3, 128), lambda i: (0, 0)),
      ],
      out_specs=[pl.BlockSpec((_NBP, 128), lambda i: (i, 0))] * nco,
      out_shape=[jax.ShapeDtypeStruct((_NPK, 128), _F32)] * nco,
  )


_layer3 = _make_layer(3)
_layer1 = _make_layer(1)


def _final_body(sref, hpr, dvr, br, out):
  sv = sref[...]
  dv = dvr[...]
  out[...] = dv * (sv[0, 0] + sv[0, 1] + 2.0 * hpr[...]) + br[...][0][None, :]


_final = pl.pallas_call(
    _final_body,
    grid=(_GRID,),
    in_specs=[
        pl.BlockSpec((1, 2, _NBP, 128), lambda i: (0, 0, i, 0)),
        pl.BlockSpec((_NBP, 128), lambda i: (i, 0)),
        pl.BlockSpec((_NBP, 128), lambda i: (i, 0)),
        pl.BlockSpec((1, 128), lambda i: (0, 0)),
    ],
    out_specs=pl.BlockSpec((_NBP, 128), lambda i: (i, 0)),
    out_shape=jax.ShapeDtypeStruct((_NPK, 128), _F32),
)


def kernel(encode_andr_channel, encode_app_id, encode_device_model,
           encode_os_version, encode_dvce_manufacturer, encode_event_sub_type,
           collector_hour, collector_minute, emb_encode_andr_channel,
           emb_encode_app_id, emb_encode_device_model, emb_encode_os_version,
           emb_encode_dvce_manufacturer, emb_encode_event_sub_type,
           emb_collector_hour, emb_collector_minute, edge_index,
           W1, b1, W2, b2, W3, b3, W4, b4, W5, b5):
  idxs = [encode_andr_channel, encode_app_id, encode_device_model,
          encode_os_version, encode_dvce_manufacturer, encode_event_sub_type,
          collector_hour, collector_minute]
  embs = [emb_encode_andr_channel, emb_encode_app_id, emb_encode_device_model,
          emb_encode_os_version, emb_encode_dvce_manufacturer,
          emb_encode_event_sub_type, emb_collector_hour, emb_collector_minute]
  dst = edge_index[1]

  tabs = _fuse_tables(*embs, W1)
  h1c0, h1c1, h1c2, deg2 = _embed_deg(*idxs, *tabs, dst)
  dv16, hp0, hp1, hp2 = _prep(deg2.reshape(2, _NPK, 128),
                              h1c0.reshape(_NPK, 128),
                              h1c1.reshape(_NPK, 128),
                              h1c2.reshape(_NPK, 128))
  hp = (hp0, hp1, hp2)

  eye8 = jnp.eye(8, dtype=_F32)
  ws = [W2, W3, W4, W5]
  bs = [b1, b2, b3, b4]
  for i in range(4):
    s = _agg3(edge_index, *[h.reshape(_NPAD, 16) for h in hp])
    nco = 3 if i < 3 else 1
    wb = ws[i].reshape(3, 16, nco, 16).transpose(0, 2, 1, 3)
    wbk = jnp.einsum("ab,cokl->coakbl", eye8, wb).reshape(3, nco, 128, 128)
    bt = jnp.tile(bs[i].reshape(3, 1, 16), (1, 8, 1)).reshape(3, 128)
    layer = _layer3 if i < 3 else _layer1
    hp = tuple(layer(s.reshape(3, 2, _NPK, 128), *hp, dv16, wbk, bt))

  s5 = _agg1(edge_index, hp[0].reshape(_NPAD, 16))
  out = _final(s5.reshape(1, 2, _NPK, 128), hp[0], dv16,
               jnp.tile(b5, 8).reshape(1, 128))
  return out.reshape(_NPAD, 16)[:_N]


# speedup vs baseline: 31.6719x; 1.1117x over previous
import functools

import jax
import jax.numpy as jnp
from jax import lax
from jax.experimental import pallas as pl
from jax.experimental.pallas import tpu as pltpu
from jax.experimental.pallas import tpu_sc as plsc

_F32 = jnp.float32
_N = 100000
_E = 1600000
_NTILES = 32
_EPT = _E // _NTILES
_EBLK = 400
_NEB = _EPT // _EBLK
_NPS = _N // 16
_ZR = 125
_VS = [100, 5000, 2000, 50, 200, 50, 24, 60]
_DS = [8, 8, 16, 10, 10, 8, 8, 8]
_OFF = [0, 8, 16, 32, 42, 52, 60, 68]
_HID = 48
_OUT = 16
_NBLK = 160
_NBLK_N = _N // _NBLK
_NB_FULL = 20

_mesh = plsc.VectorSubcoreMesh(
    core_axis_name="c", subcore_axis_name="s", num_cores=2, num_subcores=16)
_SC_PARAMS = pltpu.CompilerParams(use_tc_tiling_on_sc=False)


def _wid_cid_sid():
  cid = lax.axis_index("c")
  sid = lax.axis_index("s")
  return cid * 16 + sid, cid, sid


def _zero_fill(ref, rows):
  def body(r, _):
    ref[r, :] = jnp.zeros((16,), _F32)
    return 0
  lax.fori_loop(0, rows, body, 0)


def _zero_acc(accsh, zbuf, sid):
  def body(r, _):
    pltpu.sync_copy(zbuf, accsh.at[pl.ds(sid * _NPS + r * _ZR, _ZR)])
    return 0
  lax.fori_loop(0, 50, body, 0)


def _copy_out(accsh, dst_at, sid):
  @pl.when(sid < 15)
  def _():
    off = pl.multiple_of(sid * 6256, 8)
    pltpu.sync_copy(accsh.at[pl.ds(off, 6256)], dst_at(off, 6256))

  @pl.when(sid == 15)
  def _():
    pltpu.sync_copy(accsh.at[pl.ds(93840, 6160)], dst_at(93840, 6160))


_DBLK = 200
_NDB = _EPT // _DBLK


def _embed_deg_body(*refs):
  idxs = refs[0:8]
  tabs = refs[8:32]
  dstr = refs[32]
  h1o = refs[33:36]
  dego = refs[36]
  ib, gb, ac0, ac1, ac2, ones, eb, zbuf, accsh, sg0, sg1, si, ss = refs[37:]
  acs = (ac0, ac1, ac2)
  sgs = (sg0, sg1)
  wid, cid, sid = _wid_cid_sid()

  _zero_fill(zbuf, _ZR)
  def ones_body(r, _):
    ones[r, :] = jnp.full((16,), 1.0, _F32)
    return 0
  lax.fori_loop(0, _DBLK, ones_body, 0)

  _zero_acc(accsh, zbuf, sid)
  plsc.subcore_barrier()

  pltpu.sync_copy(dstr.at[pl.ds(wid * _EPT, _DBLK)], eb.at[0])

  def deg_body(j, _):
    m = lax.rem(j, 2)
    nm = lax.rem(j + 1, 2)

    @pl.when(j + 1 < _NDB)
    def _():
      pltpu.async_copy(dstr.at[pl.ds(wid * _EPT + (j + 1) * _DBLK, _DBLK)],
                       eb.at[nm], si)

    pltpu.async_copy(ones, accsh.at[eb.at[m]], ss, add=True)

    @pl.when(j + 1 < _NDB)
    def _():
      pltpu.make_async_copy(
          dstr.at[pl.ds(wid * _EPT + (j + 1) * _DBLK, _DBLK)],
          eb.at[nm], si).wait()

    pltpu.make_async_copy(ones, accsh.at[eb.at[m]], ss).wait()
    return 0
  lax.fori_loop(0, _NDB, deg_body, 0)
  plsc.subcore_barrier()
  _copy_out(accsh, lambda off, sz: dego.at[cid, pl.ds(off, sz)], sid)

  def gathers(f, m):
    return [pltpu.make_async_copy(tabs[f * 3 + c].at[ib.at[f % 2]],
                                  gb.at[m, c], sgs[m])
            for c in range(3)]

  nblk = jnp.where(wid < (_NBLK_N - (_NB_FULL - 1) * _NTILES),
                   _NB_FULL, _NB_FULL - 1)

  def emb_body(b, _):
    base = (wid + _NTILES * b) * _NBLK
    def zero_body(r, _):
      for c in range(3):
        acs[c][r, :] = jnp.zeros((16,), _F32)
      return 0
    lax.fori_loop(0, _NBLK, zero_body, 0)

    pltpu.sync_copy(idxs[0].at[pl.ds(base, _NBLK)], ib.at[0])
    for g in gathers(0, 0):
      g.start()
    pltpu.async_copy(idxs[1].at[pl.ds(base, _NBLK)], ib.at[1], si)
    for f in range(8):
      m = f % 2
      if f + 1 < 8:
        pltpu.make_async_copy(idxs[f + 1].at[pl.ds(base, _NBLK)],
                              ib.at[(f + 1) % 2], si).wait()
        for g in gathers(f + 1, 1 - m):
          g.start()
        if f + 2 < 8:
          pltpu.async_copy(idxs[f + 2].at[pl.ds(base, _NBLK)],
                           ib.at[f % 2], si)
      for g in gathers(f, m):
        g.wait()

      def add_body(r, _):
        for k in range(4):
          for c in range(3):
            plsc.addupdate(acs[c].at[r * 4 + k], gb[m, c, r * 4 + k])
        return 0
      lax.fori_loop(0, _NBLK // 4, add_body, 0)
    for c in range(3):
      pltpu.sync_copy(acs[c], h1o[c].at[pl.ds(base, _NBLK)])
    return 0
  lax.fori_loop(0, nblk, emb_body, 0)


_embed_deg = functools.partial(
    pl.kernel,
    out_type=[jax.ShapeDtypeStruct((102400, 16), _F32)] * 3
    + [jax.ShapeDtypeStruct((2, 102400, 16), _F32)],
    mesh=_mesh,
    scratch_types=[
        pltpu.VMEM((2, _NBLK), jnp.int32),
        pltpu.VMEM((2, 3, _NBLK, 16), _F32),
        pltpu.VMEM((_NBLK, 16), _F32),
        pltpu.VMEM((_NBLK, 16), _F32),
        pltpu.VMEM((_NBLK, 16), _F32),
        pltpu.VMEM((_DBLK, 16), _F32),
        pltpu.VMEM((2, _DBLK), jnp.int32),
        pltpu.VMEM((_ZR, 16), _F32),
        pltpu.VMEM_SHARED((_N, 16), _F32),
        pltpu.SemaphoreType.DMA,
        pltpu.SemaphoreType.DMA,
        pltpu.SemaphoreType.DMA,
        pltpu.SemaphoreType.DMA,
    ],
    compiler_params=_SC_PARAMS,
)(_embed_deg_body)


def _make_agg(nc):
  def body(*refs):
    eir = refs[0]
    hps = refs[1:1 + nc]
    so = refs[1 + nc]
    eib, rows, zbuf, accsh, sg, ss, si = refs[2 + nc:]
    wid, cid, sid = _wid_cid_sid()
    _zero_fill(zbuf, _ZR)
    ebase = wid * _EPT

    def ei_slice(j):
      return eir.at[:, pl.ds(ebase + j * _EBLK, _EBLK)]

    def gather(c, j, slot3, slot4):
      return pltpu.make_async_copy(hps[c].at[eib.at[slot4, 0]],
                                   rows.at[slot3], sg.at[slot3])

    for c in range(nc):
      _zero_acc(accsh, zbuf, sid)
      plsc.subcore_barrier()

      pltpu.sync_copy(ei_slice(0), eib.at[0])
      gather(c, 0, 0, 0).start()
      pltpu.sync_copy(ei_slice(1), eib.at[1])
      gather(c, 1, 1, 1).start()
      pltpu.async_copy(ei_slice(2), eib.at[2], si)

      def edge_body(j, _):
        m3 = lax.rem(j, 3)
        m4 = lax.rem(j, 4)
        gather(c, j, m3, m4).wait()
        pltpu.async_copy(rows.at[m3], accsh.at[eib.at[m4, 1]], ss, add=True)

        @pl.when(j + 2 < _NEB)
        def _():
          n3 = lax.rem(j + 2, 3)
          n4 = lax.rem(j + 2, 4)
          pltpu.make_async_copy(ei_slice(j + 2), eib.at[n4], si).wait()
          gather(c, j + 2, n3, n4).start()

        @pl.when(j + 3 < _NEB)
        def _():
          pltpu.async_copy(ei_slice(j + 3), eib.at[lax.rem(j + 3, 4)], si)

        pltpu.make_async_copy(rows.at[m3], accsh.at[eib.at[m4, 1]],
                              ss).wait()
        return 0
      lax.fori_loop(0, _NEB, edge_body, 0)
      plsc.subcore_barrier()
      _copy_out(accsh,
                lambda off, sz: so.at[c, cid, pl.ds(off, sz)], sid)
      plsc.subcore_barrier()

  return functools.partial(
      pl.kernel,
      out_type=jax.ShapeDtypeStruct((nc, 2, 102400, 16), _F32),
      mesh=_mesh,
      scratch_types=[
          pltpu.VMEM((4, 2, _EBLK), jnp.int32),
          pltpu.VMEM((3, _EBLK, 16), _F32),
          pltpu.VMEM((_ZR, 16), _F32),
          pltpu.VMEM_SHARED((_N, 16), _F32),
          pltpu.SemaphoreType.DMA((3,)),
          pltpu.SemaphoreType.DMA,
          pltpu.SemaphoreType.DMA,
      ],
      compiler_params=_SC_PARAMS,
  )(body)


_agg3 = _make_agg(3)
_agg1 = _make_agg(1)


def _fuse_body(*refs):
  embs = refs[0:8]
  w = refs[8]
  outs = refs[9:]
  wv = w[...]
  for f in range(8):
    ev = embs[f][...]
    for c in range(3):
      outs[f * 3 + c][...] = jnp.dot(
          ev, wv[_OFF[f]:_OFF[f] + _DS[f], c * 16:(c + 1) * 16],
          preferred_element_type=_F32)


_fuse_tables = pl.pallas_call(
    _fuse_body,
    out_shape=[jax.ShapeDtypeStruct((v, 16), _F32)
               for v in _VS for _ in range(3)],
)


_NPAD = 102400
_NPK = _NPAD // 8
_NBP = 1280
_GRID = _NPK // _NBP


def _prep_body(dref, h0r, h1r, h2r, dvo, hp0, hp1, hp2):
  v = dref[...]
  dv = lax.rsqrt(v[0] + v[1] + 2.0)
  dvo[...] = dv
  hpo = (hp0, hp1, hp2)
  for c, hr in enumerate((h0r, h1r, h2r)):
    hpo[c][...] = dv * hr[...]


_prep = pl.pallas_call(
    _prep_body,
    grid=(_GRID,),
    in_specs=[pl.BlockSpec((2, _NBP, 128), lambda i: (0, i, 0))]
    + [pl.BlockSpec((_NBP, 128), lambda i: (i, 0))] * 3,
    out_specs=[pl.BlockSpec((_NBP, 128), lambda i: (i, 0))] * 4,
    out_shape=[jax.ShapeDtypeStruct((_NPK, 128), _F32)] * 4,
)


def _make_layer(nco):
  def body(sref, hp0r, hp1r, hp2r, dvr, wbr, br, *outs):
    sv = sref[...]
    dv = dvr[...]
    wb = wbr[...]
    bv = br[...]
    hps = (hp0r[...], hp1r[...], hp2r[...])
    acts = []
    for c in range(3):
      pre = dv * (sv[c, 0] + sv[c, 1] + 2.0 * hps[c]) + bv[c][None, :]
      acts.append(jnp.maximum(pre, 0.0))
    for co in range(nco):
      h = jnp.dot(acts[0], wb[0, co], preferred_element_type=_F32)
      h = h + jnp.dot(acts[1], wb[1, co], preferred_element_type=_F32)
      h = h + jnp.dot(acts[2], wb[2, co], preferred_element_type=_F32)
      outs[co][...] = dv * h

  return pl.pallas_call(
      body,
      grid=(_GRID,),
      in_specs=[
          pl.BlockSpec((3, 2, _NBP, 128), lambda i: (0, 0, i, 0)),
          pl.BlockSpec((_NBP, 128), lambda i: (i, 0)),
          pl.BlockSpec((_NBP, 128), lambda i: (i, 0)),
          pl.BlockSpec((_NBP, 128), lambda i: (i, 0)),
          pl.BlockSpec((_NBP, 128), lambda i: (i, 0)),
          pl.BlockSpec((3, nco, 128, 128), lambda i: (0, 0, 0, 0)),
          pl.BlockSpec((3, 128), lambda i: (0, 0)),
      ],
      out_specs=[pl.BlockSpec((_NBP, 128), lambda i: (i, 0))] * nco,
      out_shape=[jax.ShapeDtypeStruct((_NPK, 128), _F32)] * nco,
  )


_layer3 = _make_layer(3)
_layer1 = _make_layer(1)


def _final_body(sref, hpr, dvr, br, out):
  sv = sref[...]
  dv = dvr[...]
  out[...] = dv * (sv[0, 0] + sv[0, 1] + 2.0 * hpr[...]) + br[...][0][None, :]


_final = pl.pallas_call(
    _final_body,
    grid=(_GRID,),
    in_specs=[
        pl.BlockSpec((1, 2, _NBP, 128), lambda i: (0, 0, i, 0)),
        pl.BlockSpec((_NBP, 128), lambda i: (i, 0)),
        pl.BlockSpec((_NBP, 128), lambda i: (i, 0)),
        pl.BlockSpec((1, 128), lambda i: (0, 0)),
    ],
    out_specs=pl.BlockSpec((_NBP, 128), lambda i: (i, 0)),
    out_shape=jax.ShapeDtypeStruct((_NPK, 128), _F32),
)


def kernel(encode_andr_channel, encode_app_id, encode_device_model,
           encode_os_version, encode_dvce_manufacturer, encode_event_sub_type,
           collector_hour, collector_minute, emb_encode_andr_channel,
           emb_encode_app_id, emb_encode_device_model, emb_encode_os_version,
           emb_encode_dvce_manufacturer, emb_encode_event_sub_type,
           emb_collector_hour, emb_collector_minute, edge_index,
           W1, b1, W2, b2, W3, b3, W4, b4, W5, b5):
  idxs = [encode_andr_channel, encode_app_id, encode_device_model,
          encode_os_version, encode_dvce_manufacturer, encode_event_sub_type,
          collector_hour, collector_minute]
  embs = [emb_encode_andr_channel, emb_encode_app_id, emb_encode_device_model,
          emb_encode_os_version, emb_encode_dvce_manufacturer,
          emb_encode_event_sub_type, emb_collector_hour, emb_collector_minute]
  dst = edge_index[1]

  tabs = _fuse_tables(*embs, W1)
  h1c0, h1c1, h1c2, deg2 = _embed_deg(*idxs, *tabs, dst)
  dv16, hp0, hp1, hp2 = _prep(deg2.reshape(2, _NPK, 128),
                              h1c0.reshape(_NPK, 128),
                              h1c1.reshape(_NPK, 128),
                              h1c2.reshape(_NPK, 128))
  hp = (hp0, hp1, hp2)

  eye8 = jnp.eye(8, dtype=_F32)
  ws = [W2, W3, W4, W5]
  bs = [b1, b2, b3, b4]
  for i in range(4):
    s = _agg3(edge_index, *[h.reshape(_NPAD, 16) for h in hp])
    nco = 3 if i < 3 else 1
    wb = ws[i].reshape(3, 16, nco, 16).transpose(0, 2, 1, 3)
    wbk = jnp.einsum("ab,cokl->coakbl", eye8, wb).reshape(3, nco, 128, 128)
    bt = jnp.tile(bs[i].reshape(3, 1, 16), (1, 8, 1)).reshape(3, 128)
    layer = _layer3 if i < 3 else _layer1
    hp = tuple(layer(s.reshape(3, 2, _NPK, 128), *hp, dv16, wbk, bt))

  s5 = _agg1(edge_index, hp[0].reshape(_NPAD, 16))
  out = _final(s5.reshape(1, 2, _NPK, 128), hp[0], dv16,
               jnp.tile(b5, 8).reshape(1, 128))
  return out.reshape(_NPAD, 16)[:_N]
